# Initial kernel scaffold; baseline (speedup 1.0000x reference)
import jax
import jax.numpy as jnp
from jax.experimental import pallas as pl
from jax.experimental.pallas import tpu as pltpu


def _tc_body(der_ref, g_ref, out_ref):
    a = pl.program_id(1)
    der = der_ref[0]
    g = g_ref[0]
    y = jnp.dot(der, g, preferred_element_type=jnp.float32)
    val = jnp.sum(y * y)

    @pl.when(a == 0)
    def _():
        out_ref[0, 0] = val

    @pl.when(a != 0)
    def _():
        out_ref[0, 0] += val


def kernel(x, der_desc_wrt_coord):
    B, A, D, K = der_desc_wrt_coord.shape
    derflat = der_desc_wrt_coord.reshape(B, A, D * K)
    g = (x[:, :, None, None] * jnp.eye(K, dtype=x.dtype)).reshape(B, D * K, K)

    TA = 32
    grid = (B, A // TA)
    out = pl.pallas_call(
        _tc_body,
        grid=grid,
        in_specs=[
            pl.BlockSpec((1, TA, D * K), lambda b, a: (b, a, 0)),
            pl.BlockSpec((1, D * K, K), lambda b, a: (b, 0, 0)),
        ],
        out_specs=pl.BlockSpec((1, 1), lambda b, a: (b, 0)),
        out_shape=jax.ShapeDtypeStruct((B, 1), jnp.float32),
        compiler_params=pltpu.CompilerParams(
            dimension_semantics=("parallel", "arbitrary"),
        ),
    )(derflat, g)
    return out.reshape(B)


# speedup vs baseline: 562.3762x; 562.3762x over previous
import jax
import jax.numpy as jnp
from jax.experimental import pallas as pl
from jax.experimental.pallas import tpu as pltpu


def _tc_body(der_ref, g_ref, out_ref):
    a = pl.program_id(1)
    der = der_ref[0]
    g = g_ref[0]
    y = jnp.dot(der, g, preferred_element_type=jnp.float32)
    val = jnp.sum(y * y).reshape(1, 1, 1)

    @pl.when(a == 0)
    def _():
        out_ref[...] = val

    @pl.when(a != 0)
    def _():
        out_ref[...] += val


def kernel(x, der_desc_wrt_coord):
    B, A, D, K = der_desc_wrt_coord.shape
    derflat = der_desc_wrt_coord.reshape(B, A, D * K)
    g = (x[:, :, None, None] * jnp.eye(K, dtype=x.dtype)).reshape(B, D * K, K)

    TA = 32
    grid = (B, A // TA)
    out = pl.pallas_call(
        _tc_body,
        grid=grid,
        in_specs=[
            pl.BlockSpec((1, TA, D * K), lambda b, a: (b, a, 0)),
            pl.BlockSpec((1, D * K, K), lambda b, a: (b, 0, 0)),
        ],
        out_specs=pl.BlockSpec((1, 1, 1), lambda b, a: (b, 0, 0)),
        out_shape=jax.ShapeDtypeStruct((B, 1, 1), jnp.float32),
        compiler_params=pltpu.CompilerParams(
            dimension_semantics=("parallel", "arbitrary"),
        ),
    )(derflat, g)
    return out.reshape(B)


# speedup vs baseline: 1915.5785x; 3.4062x over previous
import jax
import jax.numpy as jnp
from jax.experimental import pallas as pl
from jax.experimental.pallas import tpu as pltpu


def _tc_body(dp_ref, x_ref, out_ref):
    a = pl.program_id(1)
    blk = dp_ref[0]
    k3, ta, d = blk.shape
    z = blk.reshape(k3 * ta, d) * x_ref[0]
    y = jnp.sum(z, axis=1, keepdims=True)
    val = jnp.sum(y * y).reshape(1, 1, 1)

    @pl.when(a == 0)
    def _():
        out_ref[...] = val

    @pl.when(a != 0)
    def _():
        out_ref[...] += val


def kernel(x, der_desc_wrt_coord):
    B, A, D, K = der_desc_wrt_coord.shape
    dp = jnp.transpose(der_desc_wrt_coord, (0, 3, 1, 2))
    x3 = x.reshape(B, 1, D)

    TA = 64
    grid = (B, A // TA)
    out = pl.pallas_call(
        _tc_body,
        grid=grid,
        in_specs=[
            pl.BlockSpec((1, K, TA, D), lambda b, a: (b, 0, a, 0)),
            pl.BlockSpec((1, 1, D), lambda b, a: (b, 0, 0)),
        ],
        out_specs=pl.BlockSpec((1, 1, 1), lambda b, a: (b, 0, 0)),
        out_shape=jax.ShapeDtypeStruct((B, 1, 1), jnp.float32),
        compiler_params=pltpu.CompilerParams(
            dimension_semantics=("parallel", "arbitrary"),
        ),
    )(dp, x3)
    return out.reshape(B)


# speedup vs baseline: 3323.8755x; 1.7352x over previous
import jax
import jax.numpy as jnp
from jax.experimental import pallas as pl
from jax.experimental.pallas import tpu as pltpu


def _tc_body(dp_ref, x_ref, out_ref):
    a = pl.program_id(1)
    blk = dp_ref[0]
    k3, ta, d = blk.shape
    z = blk.reshape(k3 * ta, d) * x_ref[0]
    y = jnp.sum(z, axis=1, keepdims=True)
    val = jnp.sum(y * y).reshape(1, 1, 1)

    @pl.when(a == 0)
    def _():
        out_ref[...] = val

    @pl.when(a != 0)
    def _():
        out_ref[...] += val


def kernel(x, der_desc_wrt_coord):
    B, A, D, K = der_desc_wrt_coord.shape
    dp = jnp.transpose(der_desc_wrt_coord, (0, 3, 1, 2))
    x3 = x.reshape(B, 1, D)

    TA = 128
    grid = (B, A // TA)
    out = pl.pallas_call(
        _tc_body,
        grid=grid,
        in_specs=[
            pl.BlockSpec((1, K, TA, D), lambda b, a: (b, 0, a, 0)),
            pl.BlockSpec((1, 1, D), lambda b, a: (b, 0, 0)),
        ],
        out_specs=pl.BlockSpec((1, 1, 1), lambda b, a: (b, 0, 0)),
        out_shape=jax.ShapeDtypeStruct((B, 1, 1), jnp.float32),
        compiler_params=pltpu.CompilerParams(
            dimension_semantics=("parallel", "arbitrary"),
        ),
    )(dp, x3)
    return out.reshape(B)


# speedup vs baseline: 6996.3967x; 2.1049x over previous
import jax
import jax.numpy as jnp
from jax.experimental import pallas as pl
from jax.experimental.pallas import tpu as pltpu


def _tc_body(dp_ref, x_ref, out_ref):
    blk = dp_ref[...]
    nb, k3, a, d = blk.shape
    z = blk.reshape(nb, k3 * a, d) * x_ref[:, :, :]
    y = jnp.sum(z, axis=2)
    out_ref[...] = jnp.sum(y * y, axis=1).reshape(nb, 1, 1)


def kernel(x, der_desc_wrt_coord):
    B, A, D, K = der_desc_wrt_coord.shape
    dp = jnp.transpose(der_desc_wrt_coord, (0, 3, 1, 2))
    x3 = x.reshape(B, 1, D)

    NB = 4
    grid = (B // NB,)
    out = pl.pallas_call(
        _tc_body,
        grid=grid,
        in_specs=[
            pl.BlockSpec((NB, K, A, D), lambda b: (b, 0, 0, 0)),
            pl.BlockSpec((NB, 1, D), lambda b: (b, 0, 0)),
        ],
        out_specs=pl.BlockSpec((NB, 1, 1), lambda b: (b, 0, 0)),
        out_shape=jax.ShapeDtypeStruct((B, 1, 1), jnp.float32),
        compiler_params=pltpu.CompilerParams(
            dimension_semantics=("arbitrary",),
        ),
    )(dp, x3)
    return out.reshape(B)


# speedup vs baseline: 8307.0512x; 1.1873x over previous
import jax
import jax.numpy as jnp
from jax.experimental import pallas as pl
from jax.experimental.pallas import tpu as pltpu


def _tc_body(dp_ref, x_ref, out_ref):
    blk = dp_ref[...]
    nb, k3, a, d = blk.shape
    z = blk.reshape(nb, k3 * a, d) * x_ref[:, :, :]
    y = jnp.sum(z, axis=2)
    out_ref[...] = jnp.sum(y * y, axis=1).reshape(nb, 1, 1)


def kernel(x, der_desc_wrt_coord):
    B, A, D, K = der_desc_wrt_coord.shape
    dp = jnp.transpose(der_desc_wrt_coord, (0, 3, 1, 2))
    x3 = x.reshape(B, 1, D)

    NB = 8
    grid = (B // NB,)
    out = pl.pallas_call(
        _tc_body,
        grid=grid,
        in_specs=[
            pl.BlockSpec((NB, K, A, D), lambda b: (b, 0, 0, 0)),
            pl.BlockSpec((NB, 1, D), lambda b: (b, 0, 0)),
        ],
        out_specs=pl.BlockSpec((NB, 1, 1), lambda b: (b, 0, 0)),
        out_shape=jax.ShapeDtypeStruct((B, 1, 1), jnp.float32),
        compiler_params=pltpu.CompilerParams(
            dimension_semantics=("arbitrary",),
        ),
    )(dp, x3)
    return out.reshape(B)
